# baseline (speedup 1.0000x reference)
import functools

import jax
import jax.numpy as jnp
from jax import lax
from jax.experimental import pallas as pl
from jax.experimental.pallas import tpu as pltpu
from jax.experimental.pallas import tpu_sc as plsc

NUM_GRAPHS = 64
NUM_VIRTUAL = 4
N_NODES = 10000
HIDDEN = 128

NW = 32
N_PAD = 10240
ROWS_W = N_PAD // NW
CHUNK = 80
NCHUNK = ROWS_W // CHUNK


def _enc_kernel(x_ref, W_ref, b_ref, nf_ref):
    nf = jnp.maximum(jnp.dot(x_ref[...], W_ref[...]) + b_ref[...], 0.0)
    mask = lax.broadcasted_iota(jnp.int32, (N_PAD, 1), 0) < N_NODES
    nf_ref[...] = jnp.where(mask, nf, 0.0)


def _sc_segsum_body(nf_hbm, batch_hbm, zeros_hbm, out_hbm,
                    idx_v, rows_v, shared):
    cid = lax.axis_index("c")
    sid = lax.axis_index("s")
    wid = sid * 2 + cid
    base = wid * ROWS_W
    pltpu.sync_copy(batch_hbm.at[pl.ds(base, ROWS_W)], idx_v)
    pltpu.sync_copy(nf_hbm.at[pl.ds(base, ROWS_W)], rows_v)

    @pl.when(sid == 0)
    def _():
        pltpu.sync_copy(zeros_hbm, shared)

    plsc.subcore_barrier()
    for j in range(NCHUNK):
        pltpu.sync_copy(rows_v.at[pl.ds(j * CHUNK, CHUNK)],
                        shared.at[idx_v.at[pl.ds(j * CHUNK, CHUNK)]],
                        add=True)
    plsc.subcore_barrier()

    @pl.when(sid == 0)
    def _():
        pltpu.sync_copy(shared, out_hbm.at[cid])


_sc_segsum = functools.partial(
    pl.kernel,
    mesh=plsc.VectorSubcoreMesh(core_axis_name="c", subcore_axis_name="s"),
    out_type=jax.ShapeDtypeStruct((2, NUM_GRAPHS, HIDDEN), jnp.float32),
    scratch_types=[
        pltpu.VMEM((ROWS_W,), jnp.int32),
        pltpu.VMEM((ROWS_W, HIDDEN), jnp.float32),
        pltpu.VMEM_SHARED((NUM_GRAPHS, HIDDEN), jnp.float32),
    ],
)(_sc_segsum_body)


def _mlp_kernel(seg2_ref, W1_ref, b1_ref, W2_ref, b2_ref, Wp1_ref, bp1_ref,
                Wp2_ref, bp2_ref, out_ref):
    seg = (seg2_ref[0] + seg2_ref[1]) * (1.0 / NUM_VIRTUAL)
    h = jnp.maximum(jnp.dot(seg, W1_ref[...]) + b1_ref[...], 0.0)
    gf = jnp.dot(h, W2_ref[...]) + b2_ref[...]
    p = jnp.maximum(jnp.dot(gf, Wp1_ref[...]) + bp1_ref[...], 0.0)
    out_ref[...] = jnp.dot(p, Wp2_ref[...]) + bp2_ref[...]


def kernel(x, edge_index, batch, W_enc, b_enc, W1, b1, W2, b2, Wp1, bp1,
           Wp2, bp2):
    del edge_index
    x_pad = jnp.zeros((N_PAD, HIDDEN), jnp.float32).at[:N_NODES].set(x)
    batch_pad = (jnp.full((N_PAD,), NUM_GRAPHS - 1, jnp.int32)
                 .at[:N_NODES].set(batch.astype(jnp.int32)))
    zeros = jnp.zeros((NUM_GRAPHS, HIDDEN), jnp.float32)

    vmem = pl.BlockSpec(memory_space=pltpu.MemorySpace.VMEM)
    nf = pl.pallas_call(
        _enc_kernel,
        in_specs=[vmem] * 3,
        out_specs=vmem,
        out_shape=jax.ShapeDtypeStruct((N_PAD, HIDDEN), jnp.float32),
    )(x_pad, W_enc, b_enc.reshape(1, HIDDEN))

    seg2 = _sc_segsum(nf, batch_pad, zeros)

    out = pl.pallas_call(
        _mlp_kernel,
        in_specs=[vmem] * 9,
        out_specs=vmem,
        out_shape=jax.ShapeDtypeStruct((NUM_GRAPHS, 1), jnp.float32),
    )(seg2, W1, b1.reshape(1, HIDDEN), W2, b2.reshape(1, HIDDEN),
      Wp1, bp1.reshape(1, HIDDEN), Wp2, bp2.reshape(1, 1))
    return out


# speedup vs baseline: 3.1294x; 3.1294x over previous
import jax
import jax.numpy as jnp
from jax import lax
from jax.experimental import pallas as pl
from jax.experimental.pallas import tpu as pltpu

NUM_GRAPHS = 64
NUM_VIRTUAL = 4
N_NODES = 10000
HIDDEN = 128

CHUNK = 2500
NUM_CHUNKS = N_NODES // CHUNK


def _fused_kernel(x_hbm_ref, batch_ref, W_enc_ref, b_enc_ref, W1_ref, b1_ref,
                  W2_ref, b2_ref, Wp1_ref, bp1_ref, Wp2_ref, bp2_ref,
                  out_ref, xb0, xb1, xb2, xb3, sem0, sem1, sem2, sem3):
    bufs = (xb0, xb1, xb2, xb3)
    sems = (sem0, sem1, sem2, sem3)

    copies = [pltpu.make_async_copy(
        x_hbm_ref.at[pl.ds(i * CHUNK, CHUNK), :], bufs[i], sems[i])
        for i in range(NUM_CHUNKS)]
    for c in copies:
        c.start()

    acc = jnp.zeros((NUM_GRAPHS, HIDDEN), jnp.float32)
    for i in range(NUM_CHUNKS):
        copies[i].wait()
        xb = bufs[i][...]
        nf = jnp.maximum(jnp.dot(xb, W_enc_ref[...]) + b_enc_ref[...], 0.0)
        bb = batch_ref[0, pl.ds(i * CHUNK, CHUNK)]
        onehot_t = (lax.broadcasted_iota(jnp.int32, (NUM_GRAPHS, CHUNK), 0)
                    == bb[None, :]).astype(jnp.float32)
        acc = acc + jnp.dot(onehot_t, nf)

    seg = acc * (1.0 / NUM_VIRTUAL)
    h = jnp.maximum(jnp.dot(seg, W1_ref[...]) + b1_ref[...], 0.0)
    gf = jnp.dot(h, W2_ref[...]) + b2_ref[...]
    p = jnp.maximum(jnp.dot(gf, Wp1_ref[...]) + bp1_ref[...], 0.0)
    out_ref[...] = jnp.dot(p, Wp2_ref[...]) + bp2_ref[...]


def kernel(x, edge_index, batch, W_enc, b_enc, W1, b1, W2, b2, Wp1, bp1,
           Wp2, bp2):
    del edge_index
    vmem = pl.BlockSpec(memory_space=pltpu.MemorySpace.VMEM)
    out = pl.pallas_call(
        _fused_kernel,
        in_specs=[pl.BlockSpec(memory_space=pltpu.MemorySpace.HBM)]
                 + [vmem] * 11,
        out_specs=vmem,
        out_shape=jax.ShapeDtypeStruct((NUM_GRAPHS, 1), jnp.float32),
        scratch_shapes=[
            pltpu.VMEM((CHUNK, HIDDEN), jnp.float32),
            pltpu.VMEM((CHUNK, HIDDEN), jnp.float32),
            pltpu.VMEM((CHUNK, HIDDEN), jnp.float32),
            pltpu.VMEM((CHUNK, HIDDEN), jnp.float32),
            pltpu.SemaphoreType.DMA,
            pltpu.SemaphoreType.DMA,
            pltpu.SemaphoreType.DMA,
            pltpu.SemaphoreType.DMA,
        ],
    )(x, batch.reshape(1, N_NODES), W_enc, b_enc.reshape(1, HIDDEN),
      W1, b1.reshape(1, HIDDEN), W2, b2.reshape(1, HIDDEN),
      Wp1, bp1.reshape(1, HIDDEN), Wp2, bp2.reshape(1, 1))
    return out


# speedup vs baseline: 3.4308x; 1.0963x over previous
import jax
import jax.numpy as jnp
from jax import lax
from jax.experimental import pallas as pl
from jax.experimental.pallas import tpu as pltpu

NUM_GRAPHS = 64
NUM_VIRTUAL = 4
N_NODES = 10000
HIDDEN = 128


def _fused_kernel(x_ref, batch_ref, W_enc_ref, b_enc_ref, W1_ref, b1_ref,
                  W2_ref, b2_ref, Wp1_ref, bp1_ref, Wp2_ref, bp2_ref,
                  out_ref):
    nf = jnp.maximum(jnp.dot(x_ref[...], W_enc_ref[...]) + b_enc_ref[...],
                     0.0)
    bb = batch_ref[0, :]
    onehot_t = (lax.broadcasted_iota(jnp.int32, (NUM_GRAPHS, N_NODES), 0)
                == bb[None, :]).astype(jnp.float32)
    seg = jnp.dot(onehot_t, nf) * (1.0 / NUM_VIRTUAL)
    h = jnp.maximum(jnp.dot(seg, W1_ref[...]) + b1_ref[...], 0.0)
    gf = jnp.dot(h, W2_ref[...]) + b2_ref[...]
    p = jnp.maximum(jnp.dot(gf, Wp1_ref[...]) + bp1_ref[...], 0.0)
    out_ref[...] = jnp.dot(p, Wp2_ref[...]) + bp2_ref[...]


def kernel(x, edge_index, batch, W_enc, b_enc, W1, b1, W2, b2, Wp1, bp1,
           Wp2, bp2):
    del edge_index
    vmem = pl.BlockSpec(memory_space=pltpu.MemorySpace.VMEM)
    out = pl.pallas_call(
        _fused_kernel,
        in_specs=[vmem] * 12,
        out_specs=vmem,
        out_shape=jax.ShapeDtypeStruct((NUM_GRAPHS, 1), jnp.float32),
    )(x, batch.reshape(1, N_NODES), W_enc, b_enc.reshape(1, HIDDEN),
      W1, b1.reshape(1, HIDDEN), W2, b2.reshape(1, HIDDEN),
      Wp1, bp1.reshape(1, HIDDEN), Wp2, bp2.reshape(1, 1))
    return out
